# baseline (speedup 1.0000x reference)
import jax
import jax.numpy as jnp
from jax.experimental import pallas as pl
from jax.experimental.pallas import tpu as pltpu

_C = 2048
_G = 128
_H = 64
_CH = 512
_NEG = -1e30
_SCALE = 0.125


def _dot(a, b):
    return jax.lax.dot(a, b, preferred_element_type=jnp.float32)


def _dot_t(a, b):
    return jax.lax.dot_general(a, b, (((1,), (1,)), ((), ())),
                               preferred_element_type=jnp.float32)


def _tl_kernel(m_ref, ddi_ref, unr_ref, h0_ref, mrow_ref,
               wir_ref, wiz_ref, win_ref, whr_ref, whz_ref, whn_ref,
               bir_ref, biz_ref, bin_ref, bhr_ref, bhz_ref, bhn_ref,
               wq_ref, bq_ref, wk_ref, bk_ref, wv_ref, bv_ref,
               out_ref, hnew_ref):
    bf16 = jnp.bfloat16
    mm = m_ref[:]
    h0 = h0_ref[:]
    h0h = h0.astype(bf16)

    r = jax.nn.sigmoid(_dot(mm, wir_ref[:]) + bir_ref[:]
                       + _dot(h0h, whr_ref[:]) + bhr_ref[:])
    z = jax.nn.sigmoid(_dot(mm, wiz_ref[:]) + biz_ref[:]
                       + _dot(h0h, whz_ref[:]) + bhz_ref[:])
    n = jnp.tanh(_dot(mm, win_ref[:]) + bin_ref[:]
                 + r * (_dot(h0h, whn_ref[:]) + bhn_ref[:]))
    h_m1 = (1.0 - z) * n + z * h0

    ddi = ddi_ref[:]
    unr = unr_ref[:]
    qd = (_dot(ddi, wq_ref[:]) + bq_ref[:]) * _SCALE
    qu = (_dot(unr, wq_ref[:]) + bq_ref[:]) * _SCALE
    kd = _dot(ddi, wk_ref[:]) + bk_ref[:]
    ku = _dot(unr, wk_ref[:]) + bk_ref[:]
    val = _dot(mm, wv_ref[:]) + bv_ref[:]

    b2r = (mrow_ref[:, 1:2] - 1.0) * 1e30
    b3r = (mrow_ref[:, 2:3] - 1.0) * 1e30
    kdh = jnp.concatenate([kd, b2r], axis=1).astype(bf16)
    kuh = jnp.concatenate([ku, b3r], axis=1).astype(bf16)
    valh = jnp.concatenate(
        [val, jnp.ones((_C, 1), jnp.float32)], axis=1).astype(bf16)

    kmax = jnp.sqrt(jnp.maximum(
        jnp.max(jnp.sum(kd * kd, axis=1, keepdims=True)),
        jnp.max(jnp.sum(ku * ku, axis=1, keepdims=True))))

    def att_chunk(x):
        xh = jnp.concatenate(
            [x, jnp.ones((_CH, 1), jnp.float32)], axis=1).astype(bf16)
        off = jnp.sqrt(jnp.sum(x * x, axis=1, keepdims=True)) * kmax
        ld = _dot_t(xh, kdh)
        lu = _dot_t(xh, kuh)
        ed = jnp.exp(ld - off).astype(bf16)
        eu = jnp.exp(lu - off).astype(bf16)
        o = _dot(ed, valh) + _dot(eu, valh)
        return jnp.tanh(o[:, 0:_H] / o[:, _H:_H + 1])

    h2 = jnp.concatenate(
        [att_chunk(qd[c * _CH:(c + 1) * _CH, :]) for c in range(_C // _CH)],
        axis=0)
    h3 = jnp.concatenate(
        [att_chunk(qu[c * _CH:(c + 1) * _CH, :]) for c in range(_C // _CH)],
        axis=0)

    m1r = mrow_ref[:, 0:1]
    m2r = mrow_ref[:, 1:2]
    m3r = mrow_ref[:, 2:3]

    hnew = jnp.where(m1r > 0, h_m1, 0.0)
    hnew = jnp.where(m2r > 0, h2, hnew)
    hnew = jnp.where(m3r > 0, h3, hnew)
    hnew_ref[:] = hnew

    o1 = jnp.max(jnp.where(m1r > 0, h_m1, _NEG), axis=0, keepdims=True)
    o2 = jnp.max(jnp.where(m2r > 0, h2, _NEG), axis=0, keepdims=True)
    o3 = jnp.max(jnp.where(m3r > 0, h3, _NEG), axis=0, keepdims=True)
    out_ref[:] = jnp.maximum(o1, jnp.maximum(o2, o3))


def kernel(m_embeddings, divided, ddi_embeddings, unrelated_embeddings,
           hidden_state, W_ih, b_ih, W_hh, b_hh, Wq, bq, Wk, bk, Wv, bv):
    f32 = jnp.float32
    bf16 = jnp.bfloat16
    mrow = (divided > 0).astype(f32)

    wih_t = W_ih.T.astype(bf16)
    whh_t = W_hh.T.astype(bf16)
    wir, wiz, win = (wih_t[:, :_H], wih_t[:, _H:2 * _H], wih_t[:, 2 * _H:])
    whr, whz, whn = (whh_t[:, :_H], whh_t[:, _H:2 * _H], whh_t[:, 2 * _H:])
    bir, biz, bin_ = (b_ih[None, :_H], b_ih[None, _H:2 * _H],
                      b_ih[None, 2 * _H:])
    bhr, bhz, bhn = (b_hh[None, :_H], b_hh[None, _H:2 * _H],
                     b_hh[None, 2 * _H:])

    out, h_new = pl.pallas_call(
        _tl_kernel,
        out_shape=(jax.ShapeDtypeStruct((1, _H), f32),
                   jax.ShapeDtypeStruct((_C, _H), f32)),
        compiler_params=pltpu.CompilerParams(
            vmem_limit_bytes=112 * 1024 * 1024),
    )(m_embeddings.astype(bf16), ddi_embeddings.astype(bf16),
      unrelated_embeddings.astype(bf16), hidden_state,
      mrow,
      wir, wiz, win, whr, whz, whn,
      bir, biz, bin_, bhr, bhz, bhn,
      Wq.T.astype(bf16), bq[None, :], Wk.T.astype(bf16), bk[None, :],
      Wv.T.astype(bf16), bv[None, :])

    return out.reshape(_H), h_new


# speedup vs baseline: 1.1794x; 1.1794x over previous
import jax
import jax.numpy as jnp
from jax.experimental import pallas as pl
from jax.experimental.pallas import tpu as pltpu

_C = 2048
_G = 128
_H = 64
_CH = 512
_NEG = -1e30
_SCALE = 0.125


def _dot(a, b):
    return jax.lax.dot(a, b, preferred_element_type=jnp.float32)


def _dot_t(a, b):
    return jax.lax.dot_general(a, b, (((1,), (1,)), ((), ())),
                               preferred_element_type=jnp.float32)


def _tl_kernel(m_ref, ddi_ref, unr_ref, h0_ref, mrow_ref,
               wir_ref, wiz_ref, win_ref, whr_ref, whz_ref, whn_ref,
               bir_ref, biz_ref, bin_ref, bhr_ref, bhz_ref, bhn_ref,
               wq_ref, bq_ref, wk_ref, bk_ref, wv_ref, bv_ref,
               out_ref, hnew_ref):
    bf16 = jnp.bfloat16
    mm = m_ref[:]
    h0 = h0_ref[:]

    r = jax.nn.sigmoid(_dot(mm, wir_ref[:]) + bir_ref[:]
                       + _dot(h0, whr_ref[:]) + bhr_ref[:])
    z = jax.nn.sigmoid(_dot(mm, wiz_ref[:]) + biz_ref[:]
                       + _dot(h0, whz_ref[:]) + bhz_ref[:])
    n = jnp.tanh(_dot(mm, win_ref[:]) + bin_ref[:]
                 + r * (_dot(h0, whn_ref[:]) + bhn_ref[:]))
    h_m1 = (1.0 - z) * n + z * h0

    ddi = ddi_ref[:]
    unr = unr_ref[:]
    qd = (_dot(ddi, wq_ref[:]) + bq_ref[:]) * _SCALE
    qu = (_dot(unr, wq_ref[:]) + bq_ref[:]) * _SCALE
    kd = _dot(ddi, wk_ref[:]) + bk_ref[:]
    ku = _dot(unr, wk_ref[:]) + bk_ref[:]
    val = _dot(mm, wv_ref[:]) + bv_ref[:]

    b2r = (mrow_ref[:, 1:2] - 1.0) * 1e30
    b3r = (mrow_ref[:, 2:3] - 1.0) * 1e30
    kdh = jnp.concatenate([kd, b2r], axis=1).astype(bf16)
    kuh = jnp.concatenate([ku, b3r], axis=1).astype(bf16)
    valh = jnp.concatenate(
        [val, jnp.ones((_C, 1), jnp.float32)], axis=1).astype(bf16)

    kmax = jnp.sqrt(jnp.maximum(
        jnp.max(jnp.sum(kd * kd, axis=1, keepdims=True)),
        jnp.max(jnp.sum(ku * ku, axis=1, keepdims=True))))

    def att_chunk(x):
        xh = jnp.concatenate(
            [x, jnp.ones((_CH, 1), jnp.float32)], axis=1).astype(bf16)
        off = jnp.sqrt(jnp.sum(x * x, axis=1, keepdims=True)) * kmax
        ld = _dot_t(xh, kdh)
        lu = _dot_t(xh, kuh)
        ed = jnp.exp((ld - off).astype(bf16))
        eu = jnp.exp((lu - off).astype(bf16))
        o = _dot(ed, valh) + _dot(eu, valh)
        return jnp.tanh(o[:, 0:_H] / o[:, _H:_H + 1])

    h2 = jnp.concatenate(
        [att_chunk(qd[c * _CH:(c + 1) * _CH, :]) for c in range(_C // _CH)],
        axis=0)
    h3 = jnp.concatenate(
        [att_chunk(qu[c * _CH:(c + 1) * _CH, :]) for c in range(_C // _CH)],
        axis=0)

    m1r = mrow_ref[:, 0:1]
    m2r = mrow_ref[:, 1:2]
    m3r = mrow_ref[:, 2:3]

    hnew = jnp.where(m1r > 0, h_m1, 0.0)
    hnew = jnp.where(m2r > 0, h2, hnew)
    hnew = jnp.where(m3r > 0, h3, hnew)
    hnew_ref[:] = hnew

    o1 = jnp.max(jnp.where(m1r > 0, h_m1, _NEG), axis=0, keepdims=True)
    o2 = jnp.max(jnp.where(m2r > 0, h2, _NEG), axis=0, keepdims=True)
    o3 = jnp.max(jnp.where(m3r > 0, h3, _NEG), axis=0, keepdims=True)
    out_ref[:] = jnp.maximum(o1, jnp.maximum(o2, o3))


def kernel(m_embeddings, divided, ddi_embeddings, unrelated_embeddings,
           hidden_state, W_ih, b_ih, W_hh, b_hh, Wq, bq, Wk, bk, Wv, bv):
    f32 = jnp.float32
    mrow = (divided > 0).astype(f32)

    wih_t = W_ih.T
    whh_t = W_hh.T
    wir, wiz, win = (wih_t[:, :_H], wih_t[:, _H:2 * _H], wih_t[:, 2 * _H:])
    whr, whz, whn = (whh_t[:, :_H], whh_t[:, _H:2 * _H], whh_t[:, 2 * _H:])
    bir, biz, bin_ = (b_ih[None, :_H], b_ih[None, _H:2 * _H],
                      b_ih[None, 2 * _H:])
    bhr, bhz, bhn = (b_hh[None, :_H], b_hh[None, _H:2 * _H],
                     b_hh[None, 2 * _H:])

    out, h_new = pl.pallas_call(
        _tl_kernel,
        out_shape=(jax.ShapeDtypeStruct((1, _H), f32),
                   jax.ShapeDtypeStruct((_C, _H), f32)),
        compiler_params=pltpu.CompilerParams(
            vmem_limit_bytes=112 * 1024 * 1024),
    )(m_embeddings, ddi_embeddings, unrelated_embeddings, hidden_state,
      mrow,
      wir, wiz, win, whr, whz, whn,
      bir, biz, bin_, bhr, bhz, bhn,
      Wq.T, bq[None, :], Wk.T, bk[None, :], Wv.T, bv[None, :])

    return out.reshape(_H), h_new
